# baseline (device time: 353241 ns/iter reference)
import jax
import jax.numpy as jnp
from jax import lax
from jax.experimental import pallas as pl
from jax.experimental.pallas import tpu as pltpu

N_DEV = 8
SCALE = 0.08838834764831843
BLK = 64
QT = 512
KT = 512
HQ = 8
DH = 128

NC = 8
CH = 256
TOTC = 2 * NC

TREE_CHILDREN = [
    {0: [1], 1: [2, 5], 2: [3, 6], 5: [4], 6: [7]},
    {0: [3], 3: [7, 2], 7: [4, 6], 2: [1], 6: [5]},
    {0: [4], 4: [5, 7], 5: [1, 6], 7: [3], 6: [2]},
]

STAGE_DEPS = [
    [0, 1, NC + 0, NC + 1],
    [2, 3, NC + 2, NC + 3],
    [4, 5, NC + 4, NC + 5],
    [6, 7, NC + 6, NC + 7],
]


def _tree_depths():
    depths = []
    for ch in TREE_CHILDREN:
        d = {0: 0}
        frontier = [0]
        while frontier:
            nxt = []
            for u in frontier:
                for v in ch.get(u, []):
                    d[v] = d[u] + 1
                    nxt.append(v)
            frontier = nxt
        depths.append(d)
    return depths


TREE_DEPTH = _tree_depths()


def _stage_order(p, stage):
    return sorted(
        STAGE_DEPS[stage],
        key=lambda c: (c // 3 + TREE_DEPTH[c % 3][p], c // 3, c),
    )


def _fused_body(
    x_ref,
    wq_ref,
    k_hbm,
    v_hbm,
    wo_ref,
    out_ref,
    kout,
    vout,
    q_ref,
    s_ref,
    ctx_ref,
    r_sems,
    s0_sems,
    s1_sems,
    loc_sems,
):
    my = lax.axis_index("i")

    def chunk_ref(kb, vb, c):
        base = kb if c < NC else vb
        return base.at[pl.ds((c % NC) * CH, CH), :]

    def desc(c, child, slot, from_input):
        kb = k_hbm if from_input else kout
        vb = v_hbm if from_input else vout
        sem = s0_sems if slot == 0 else s1_sems
        return pltpu.make_async_remote_copy(
            src_ref=chunk_ref(kb, vb, c),
            dst_ref=chunk_ref(kout, vout, c),
            send_sem=sem.at[c],
            recv_sem=r_sems.at[c],
            device_id=(child,),
            device_id_type=pl.DeviceIdType.MESH,
        )

    @pl.when(my == 0)
    def _():
        for c in range(TOTC):
            for slot, child in enumerate(TREE_CHILDREN[c % 3].get(0, [])):
                desc(c, child, slot, True).start()
        pltpu.make_async_copy(k_hbm, kout, loc_sems.at[0]).start()
        pltpu.make_async_copy(v_hbm, vout, loc_sems.at[1]).start()
        pltpu.make_async_copy(k_hbm, kout, loc_sems.at[0]).wait()
        pltpu.make_async_copy(v_hbm, vout, loc_sems.at[1]).wait()

    def compute_tile(stage):
        rows = pl.ds(stage * QT, QT)
        q_ref[...] = jnp.dot(
            x_ref[0, rows, :], wq_ref[...], preferred_element_type=jnp.float32
        )
        width = (stage + 1) * KT

        def head_body(h, _):
            hcols = pl.ds(h * DH, DH)
            q = q_ref[:, hcols]
            for kt in range(stage + 1):
                sc = lax.dot_general(
                    q,
                    kout[pl.ds(kt * KT, KT), hcols],
                    (((1,), (1,)), ((), ())),
                    preferred_element_type=jnp.float32,
                )
                sc = sc * SCALE
                if kt == stage:
                    row = lax.broadcasted_iota(jnp.int32, sc.shape, 0)
                    col = lax.broadcasted_iota(jnp.int32, sc.shape, 1)
                    sc = jnp.where((col // BLK) <= (row // BLK), sc, -1e9)
                s_ref[:, pl.ds(kt * KT, KT)] = sc
            s = s_ref[:, pl.ds(0, width)]
            m = jnp.max(s, axis=1, keepdims=True)
            w = jnp.exp(s - m)
            w = w / jnp.sum(w, axis=1, keepdims=True)
            s_ref[:, pl.ds(0, width)] = w
            acc = jnp.dot(
                s_ref[:, pl.ds(0, KT)],
                vout[pl.ds(0, KT), hcols],
                preferred_element_type=jnp.float32,
            )
            for kt in range(1, stage + 1):
                acc = acc + jnp.dot(
                    s_ref[:, pl.ds(kt * KT, KT)],
                    vout[pl.ds(kt * KT, KT), hcols],
                    preferred_element_type=jnp.float32,
                )
            ctx_ref[:, hcols] = acc
            return 0

        lax.fori_loop(0, HQ, head_body, 0)
        out_ref[0, rows, :] = jnp.dot(
            ctx_ref[...], wo_ref[...], preferred_element_type=jnp.float32
        )

    for stage in range(len(STAGE_DEPS)):
        for p in range(1, N_DEV):

            @pl.when(my == p)
            def _(p=p, stage=stage):
                for c in _stage_order(p, stage):
                    desc(c, 0, 0, False).wait_recv()
                    for slot, child in enumerate(TREE_CHILDREN[c % 3].get(p, [])):
                        desc(c, child, slot, False).start()

        compute_tile(stage)

    @pl.when(my == 0)
    def _():
        for c in range(TOTC):
            for slot, child in enumerate(TREE_CHILDREN[c % 3].get(0, [])):
                desc(c, child, slot, True).wait_send()

    for p in range(1, N_DEV):
        sends = [
            (c, slot, child)
            for c in range(TOTC)
            for slot, child in enumerate(TREE_CHILDREN[c % 3].get(p, []))
        ]
        if not sends:
            continue

        @pl.when(my == p)
        def _(p=p, sends=sends):
            for c, slot, child in sends:
                desc(c, child, slot, False).wait_send()


def kernel(x, Wq, K_ext, V_ext, Wo):
    B, Sq, Dm = x.shape
    _, Skv, Hq, Dh = K_ext.shape

    k2 = K_ext.reshape(Skv, Hq * Dh)
    v2 = V_ext.reshape(Skv, Hq * Dh)

    out = pl.pallas_call(
        _fused_body,
        out_shape=jax.ShapeDtypeStruct((B, Sq, Dm), jnp.float32),
        in_specs=[
            pl.BlockSpec(memory_space=pltpu.VMEM),
            pl.BlockSpec(memory_space=pltpu.VMEM),
            pl.BlockSpec(memory_space=pl.ANY),
            pl.BlockSpec(memory_space=pl.ANY),
            pl.BlockSpec(memory_space=pltpu.VMEM),
        ],
        out_specs=pl.BlockSpec(memory_space=pltpu.VMEM),
        scratch_shapes=[
            pltpu.VMEM((Skv, Hq * Dh), jnp.float32),
            pltpu.VMEM((Skv, Hq * Dh), jnp.float32),
            pltpu.VMEM((QT, Hq * Dh), jnp.float32),
            pltpu.VMEM((QT, Skv), jnp.float32),
            pltpu.VMEM((QT, Hq * Dh), jnp.float32),
            pltpu.SemaphoreType.DMA((TOTC,)),
            pltpu.SemaphoreType.DMA((TOTC,)),
            pltpu.SemaphoreType.DMA((TOTC,)),
            pltpu.SemaphoreType.DMA((2,)),
        ],
        compiler_params=pltpu.CompilerParams(
            vmem_limit_bytes=100 * 1024 * 1024,
        ),
    )(x, Wq, k2, v2, Wo)
    return out
